# baseline (speedup 1.0000x reference)
import functools

import jax
import jax.numpy as jnp
from jax import lax
from jax.experimental import pallas as pl
from jax.experimental.pallas import tpu as pltpu
from jax.experimental.pallas import tpu_sc as plsc

_NS = 16
_K = 128
_G = 8


@functools.cache
def _make_sc_agg(B, N, DIN, NGRP):
    n_pad = -(-N // (_NS * 128)) * (_NS * 128)
    mesh = plsc.VectorSubcoreMesh(
        core_axis_name="c", subcore_axis_name="s", num_cores=B)
    rows_t = n_pad // _NS
    rc = 128
    nq = DIN // 16

    @functools.partial(
        pl.kernel,
        out_type=jax.ShapeDtypeStruct((B * n_pad, DIN), jnp.float32),
        mesh=mesh,
        scratch_types=[
            [pltpu.VMEM((2, _G, _K), jnp.int32)] * 2,
            pltpu.VMEM((2, _G, _K), jnp.float32),
            [pltpu.VMEM((_K, DIN), jnp.float32)] * 2,
            pltpu.VMEM_SHARED((n_pad, DIN), jnp.float32),
            [pltpu.SemaphoreType.DMA] * 2,
            [pltpu.SemaphoreType.DMA] * 2,
            pltpu.SemaphoreType.DMA,
        ],
    )
    def sc_agg(x_hbm, src_hbm, dst_hbm, ew_hbm, out_hbm,
               idx_v, ew_v, rows, agg_sh, gsem, ssem, isem):
        src_v, dst_v = idx_v
        c = lax.axis_index("c")
        s = lax.axis_index("s")

        @plsc.parallel_loop(0, rc)
        def _zero(e):
            for q in range(nq):
                rows[0][e, pl.ds(q * 16, 16)] = jnp.zeros((16,), jnp.float32)

        base = s * rows_t
        for k in range(rows_t // rc):
            pltpu.sync_copy(rows[0].at[pl.ds(0, rc)],
                            agg_sh.at[pl.ds(base + k * rc, rc)])
        plsc.subcore_barrier()

        def stage_idx(im, sl, sync=False):
            copy = pltpu.sync_copy if sync else (
                lambda a, b: pltpu.async_copy(a, b, isem))
            copy(src_hbm.at[c, s, im], src_v.at[sl])
            copy(dst_hbm.at[c, s, im], dst_v.at[sl])
            copy(ew_hbm.at[c, s, im], ew_v.at[sl])

        def wait_idx(sl):
            pltpu.make_async_copy(src_hbm.at[c, s, 0], src_v.at[sl],
                                  isem).wait()
            pltpu.make_async_copy(dst_hbm.at[c, s, 0], dst_v.at[sl],
                                  isem).wait()
            pltpu.make_async_copy(ew_hbm.at[c, s, 0], ew_v.at[sl],
                                  isem).wait()

        def issue_gather(sl, hh, buf):
            pltpu.async_copy(x_hbm.at[src_v.at[sl, hh]], rows[buf],
                             gsem[buf])

        def wait_gather(buf):
            pltpu.make_async_copy(x_hbm.at[src_v.at[0, 0]], rows[buf],
                                  gsem[buf]).wait()

        def issue_scatter(sl, hh, buf):
            pltpu.async_copy(rows[buf], agg_sh.at[dst_v.at[sl, hh]],
                             ssem[buf], add=True)

        def wait_scatter(buf):
            pltpu.make_async_copy(rows[buf], agg_sh.at[dst_v.at[0, 0]],
                                  ssem[buf]).wait()

        def scale(sl, hh, buf):
            rb = rows[buf]

            @plsc.parallel_loop(0, _K // 16)
            def _scale(g):
                wv = ew_v[sl, hh, pl.ds(g * 16, 16)]
                for l in range(16):
                    w = wv[l]
                    e = g * 16 + l
                    for q in range(nq):
                        qs = pl.ds(q * 16, 16)
                        rb[e, qs] = rb[e, qs] * w

        def group(im, sl, stage_next, pf_next):
            nxt = 1 - sl
            if stage_next:
                stage_idx(im + 1, nxt)

            def qbody(q, carry):
                hh = 2 * q
                wait_gather(0)
                scale(sl, hh, 0)
                issue_scatter(sl, hh, 0)
                wait_gather(1)
                scale(sl, hh + 1, 1)
                issue_scatter(sl, hh + 1, 1)
                wait_scatter(0)

                @pl.when(q < _G // 2 - 1)
                def _pf0():
                    issue_gather(sl, hh + 2, 0)
                if pf_next:
                    @pl.when(q == _G // 2 - 1)
                    def _pn0():
                        wait_idx(nxt)
                        issue_gather(nxt, 0, 0)
                wait_scatter(1)

                @pl.when(q < _G // 2 - 1)
                def _pf1():
                    issue_gather(sl, hh + 3, 1)
                if pf_next:
                    @pl.when(q == _G // 2 - 1)
                    def _pn1():
                        issue_gather(nxt, 1, 1)
                return carry
            lax.fori_loop(0, _G // 2, qbody, 0)

        stage_idx(0, 0, sync=True)
        issue_gather(0, 0, 0)
        issue_gather(0, 1, 1)
        group(0, 0, True, True)

        def pair(p, carry):
            im = 1 + 2 * p
            group(im, 1, True, True)
            group(im + 1, 0, True, True)
            return carry
        lax.fori_loop(0, (NGRP - 2) // 2, pair, 0)

        group(NGRP - 1, 1, False, False)
        plsc.subcore_barrier()

        out_base = c * n_pad + base
        for k in range(rows_t // rc):
            pltpu.sync_copy(agg_sh.at[pl.ds(base + k * rc, rc)],
                            rows[0].at[pl.ds(0, rc)])
            pltpu.sync_copy(rows[0].at[pl.ds(0, rc)],
                            out_hbm.at[pl.ds(out_base + k * rc, rc)])

    return sc_agg


def _tc_body(a_ref, w_ref, b_ref, o_ref):
    acc = jnp.dot(a_ref[0], w_ref[...], preferred_element_type=jnp.float32)
    o_ref[0] = jnp.maximum(acc + b_ref[...], 0.0)


@functools.cache
def _make_tc_matmul(B, N, n_pad, DIN, DOUT, bm):
    return pl.pallas_call(
        _tc_body,
        out_shape=jax.ShapeDtypeStruct((B, N, DOUT), jnp.float32),
        grid=(B, N // bm),
        in_specs=[
            pl.BlockSpec((1, bm, DIN), lambda bb, i: (bb, i, 0)),
            pl.BlockSpec((DIN, DOUT), lambda bb, i: (0, 0)),
            pl.BlockSpec((1, DOUT), lambda bb, i: (0, 0)),
        ],
        out_specs=pl.BlockSpec((1, bm, DOUT), lambda bb, i: (bb, i, 0)),
    )


def kernel(inputs, edge_index, edge_weight, W, b):
    B, N, DIN = inputs.shape
    E = edge_weight.shape[1]
    DOUT = W.shape[1]
    ngrp = 2 * (-(-E // (_NS * _G * _K * 2)))
    epad = _NS * ngrp * _G * _K

    src = edge_index[:, 0, :].astype(jnp.int32)
    dst = edge_index[:, 1, :].astype(jnp.int32)
    ew = edge_weight.astype(jnp.float32)
    pad = epad - E
    if pad:
        fill = jnp.arange(pad, dtype=jnp.int32) % N
        src = jnp.concatenate([src, jnp.broadcast_to(fill, (B, pad))], axis=1)
        dst = jnp.concatenate([dst, jnp.broadcast_to(fill, (B, pad))], axis=1)
        ew = jnp.concatenate([ew, jnp.zeros((B, pad), jnp.float32)], axis=1)

    src = src + (jnp.arange(B, dtype=jnp.int32) * N)[:, None]
    src = src.reshape(B, _NS, ngrp, _G, _K)
    dst = dst.reshape(B, _NS, ngrp, _G, _K)
    ew = ew.reshape(B, _NS, ngrp, _G, _K)
    x_flat = inputs.reshape(B * N, DIN)

    n_pad = -(-N // (_NS * 128)) * (_NS * 128)
    agg = _make_sc_agg(B, N, DIN, ngrp)(x_flat, src, dst, ew)
    out = _make_tc_matmul(B, N, n_pad, DIN, DOUT, 2000)(
        agg.reshape(B, n_pad, DIN), W, b.reshape(1, DOUT))
    return out


# speedup vs baseline: 1.1502x; 1.1502x over previous
import functools

import jax
import jax.numpy as jnp
from jax import lax
from jax.experimental import pallas as pl
from jax.experimental.pallas import tpu as pltpu
from jax.experimental.pallas import tpu_sc as plsc

_NS = 16
_K = 112
_NB = 3


@functools.cache
def _make_sc_agg(B, N, DIN, NGRP):
    n_pad = -(-N // (_NS * 128)) * (_NS * 128)
    mesh = plsc.VectorSubcoreMesh(
        core_axis_name="c", subcore_axis_name="s", num_cores=B)
    rows_t = n_pad // _NS
    rc = 80
    nq = DIN // 16

    @functools.partial(
        pl.kernel,
        out_type=jax.ShapeDtypeStruct((B * n_pad, DIN), jnp.float32),
        mesh=mesh,
        scratch_types=[
            [pltpu.VMEM((2, _NB, _K), jnp.int32)] * 2,
            pltpu.VMEM((2, _NB, _K), jnp.float32),
            [pltpu.VMEM((_K, DIN), jnp.float32)] * _NB,
            pltpu.VMEM_SHARED((n_pad, DIN), jnp.float32),
            [pltpu.SemaphoreType.DMA] * _NB,
            [pltpu.SemaphoreType.DMA] * _NB,
            pltpu.SemaphoreType.DMA,
        ],
    )
    def sc_agg(x_hbm, src_hbm, dst_hbm, ew_hbm, out_hbm,
               idx_v, ew_v, rows, agg_sh, gsem, ssem, isem):
        src_v, dst_v = idx_v
        c = lax.axis_index("c")
        s = lax.axis_index("s")
        base = s * rows_t

        def stage_idx(im, sl, sync=False):
            if sync:
                pltpu.sync_copy(src_hbm.at[c, s, im], src_v.at[sl])
                pltpu.sync_copy(dst_hbm.at[c, s, im], dst_v.at[sl])
                pltpu.sync_copy(ew_hbm.at[c, s, im], ew_v.at[sl])
            else:
                pltpu.async_copy(src_hbm.at[c, s, im], src_v.at[sl], isem)
                pltpu.async_copy(dst_hbm.at[c, s, im], dst_v.at[sl], isem)
                pltpu.async_copy(ew_hbm.at[c, s, im], ew_v.at[sl], isem)

        def wait_idx(sl):
            pltpu.make_async_copy(src_hbm.at[c, s, 0], src_v.at[sl],
                                  isem).wait()
            pltpu.make_async_copy(dst_hbm.at[c, s, 0], dst_v.at[sl],
                                  isem).wait()
            pltpu.make_async_copy(ew_hbm.at[c, s, 0], ew_v.at[sl],
                                  isem).wait()

        def issue_gather(sl, b, buf):
            pltpu.async_copy(x_hbm.at[src_v.at[sl, b]], rows[buf], gsem[buf])

        def wait_gather(buf):
            pltpu.make_async_copy(x_hbm.at[src_v.at[0, 0]], rows[buf],
                                  gsem[buf]).wait()

        def issue_scatter(sl, b):
            pltpu.async_copy(rows[b], agg_sh.at[dst_v.at[sl, b]], ssem[b],
                             add=True)

        def wait_scatter(b):
            pltpu.make_async_copy(rows[b], agg_sh.at[dst_v.at[0, 0]],
                                  ssem[b]).wait()

        def scale(sl, b):
            rb = rows[b]

            @plsc.parallel_loop(0, _K // 16, unroll=2)
            def _scale(g):
                wv = ew_v[sl, b, pl.ds(g * 16, 16)]
                for l in range(16):
                    w = wv[l]
                    e = g * 16 + l
                    for q in range(nq):
                        qs = pl.ds(q * 16, 16)
                        rb[e, qs] = rb[e, qs] * w

        def chunk_body(im, sl, b, first_group, last_group):
            nxt = 1 - sl
            wait_gather(b)
            scale(sl, b)
            if not (first_group and b == 0):
                wait_scatter((b + _NB - 1) % _NB)
            if b == 0 and not last_group:
                stage_idx(im + 1, nxt)
            if b == 0:
                issue_gather(sl, 2, 2)
            elif not last_group:
                if b == 1:
                    wait_idx(nxt)
                issue_gather(nxt, b - 1, b - 1)
            issue_scatter(sl, b)

        stage_idx(0, 0, sync=True)
        issue_gather(0, 0, 0)
        issue_gather(0, 1, 1)

        @plsc.parallel_loop(0, rc)
        def _zero(e):
            for q in range(nq):
                rows[2][e, pl.ds(q * 16, 16)] = jnp.zeros((16,), jnp.float32)

        for k in range(rows_t // rc):
            pltpu.sync_copy(rows[2].at[pl.ds(0, rc)],
                            agg_sh.at[pl.ds(base + k * rc, rc)])
        plsc.subcore_barrier()

        for b in range(_NB):
            chunk_body(0, 0, b, True, False)

        def pair(p, carry):
            im = 1 + 2 * p
            for b in range(_NB):
                chunk_body(im, 1, b, False, False)
            for b in range(_NB):
                chunk_body(im + 1, 0, b, False, False)
            return carry
        lax.fori_loop(0, (NGRP - 2) // 2, pair, 0)

        for b in range(_NB):
            chunk_body(NGRP - 1, 1, b, False, True)
        wait_scatter(_NB - 1)
        plsc.subcore_barrier()

        out_base = c * n_pad + base
        nko = rows_t // rc
        for k in range(nko):
            rb = rows[k % 2].at[pl.ds(0, rc)]
            osl = pl.ds(out_base + k * rc, rc)
            if k >= 2:
                pltpu.make_async_copy(
                    rb, out_hbm.at[pl.ds(out_base + (k - 2) * rc, rc)],
                    gsem[k % 2]).wait()
            pltpu.sync_copy(agg_sh.at[pl.ds(base + k * rc, rc)], rb)
            pltpu.async_copy(rb, out_hbm.at[osl], gsem[k % 2])
        for k in (nko - 2, nko - 1):
            pltpu.make_async_copy(
                rows[k % 2].at[pl.ds(0, rc)],
                out_hbm.at[pl.ds(out_base + k * rc, rc)], gsem[k % 2]).wait()

    return sc_agg


def _tc_body(a_ref, w_ref, b_ref, o_ref):
    acc = jnp.dot(a_ref[0], w_ref[...], preferred_element_type=jnp.float32)
    o_ref[0] = jnp.maximum(acc + b_ref[...], 0.0)


@functools.cache
def _make_tc_matmul(B, N, n_pad, DIN, DOUT, bm):
    return pl.pallas_call(
        _tc_body,
        out_shape=jax.ShapeDtypeStruct((B, N, DOUT), jnp.float32),
        grid=(B, N // bm),
        in_specs=[
            pl.BlockSpec((1, bm, DIN), lambda bb, i: (bb, i, 0)),
            pl.BlockSpec((DIN, DOUT), lambda bb, i: (0, 0)),
            pl.BlockSpec((1, DOUT), lambda bb, i: (0, 0)),
        ],
        out_specs=pl.BlockSpec((1, bm, DOUT), lambda bb, i: (bb, i, 0)),
    )


def kernel(inputs, edge_index, edge_weight, W, b):
    B, N, DIN = inputs.shape
    E = edge_weight.shape[1]
    DOUT = W.shape[1]
    ngrp = 2 * (-(-E // (_NS * _NB * _K * 2)))
    epad = _NS * ngrp * _NB * _K

    src = edge_index[:, 0, :].astype(jnp.int32)
    dst = edge_index[:, 1, :].astype(jnp.int32)
    ew = edge_weight.astype(jnp.float32)
    pad = epad - E
    if pad:
        fill = jnp.arange(pad, dtype=jnp.int32) % N
        src = jnp.concatenate([src, jnp.broadcast_to(fill, (B, pad))], axis=1)
        dst = jnp.concatenate([dst, jnp.broadcast_to(fill, (B, pad))], axis=1)
        ew = jnp.concatenate([ew, jnp.zeros((B, pad), jnp.float32)], axis=1)

    src = src + (jnp.arange(B, dtype=jnp.int32) * N)[:, None]
    src = src.reshape(B, _NS, ngrp, _NB, _K)
    dst = dst.reshape(B, _NS, ngrp, _NB, _K)
    ew = ew.reshape(B, _NS, ngrp, _NB, _K)
    x_flat = inputs.reshape(B * N, DIN)

    n_pad = -(-N // (_NS * 128)) * (_NS * 128)
    agg = _make_sc_agg(B, N, DIN, ngrp)(x_flat, src, dst, ew)
    out = _make_tc_matmul(B, N, n_pad, DIN, DOUT, 2000)(
        agg.reshape(B, n_pad, DIN), W, b.reshape(1, DOUT))
    return out
